# baseline (speedup 1.0000x reference)
import jax
import jax.numpy as jnp
from jax import lax
from jax.experimental import pallas as pl
from jax.experimental.pallas import tpu as pltpu
from jax.experimental.pallas import tpu_sc as plsc

B = 16384
D = 64
NC = 2
NS = 16
NW = NC * NS
RPW = B // NW
GPW = RPW // 16
IDX_CHUNK = 128
NCHUNK = RPW // IDX_CHUNK


def _mf_body(user_hbm, pos_hbm, neg_hbm, user_table, item_table,
             wb_hbm, bv_hbm, outp_hbm, outn_hbm,
             uidx_v, pidx_v, nidx_v, urows_v, prows_v, nrows_v,
             wb_v, bv_v, tsp_v, tsn_v, outp_v, outn_v, sem):
    wid = lax.axis_index("s") * NC + lax.axis_index("c")
    base = wid * NCHUNK

    pltpu.sync_copy(user_hbm.at[pl.ds(base, NCHUNK)], uidx_v)
    pltpu.sync_copy(pos_hbm.at[pl.ds(base, NCHUNK)], pidx_v)
    pltpu.sync_copy(neg_hbm.at[pl.ds(base, NCHUNK)], nidx_v)
    pltpu.sync_copy(wb_hbm, wb_v)
    pltpu.sync_copy(bv_hbm, bv_v)

    copies = []
    for j in range(NCHUNK):
        sl = pl.ds(j * IDX_CHUNK, IDX_CHUNK)
        copies.append(pltpu.async_copy(user_table.at[uidx_v.at[j]],
                                       urows_v.at[sl], sem))
        copies.append(pltpu.async_copy(item_table.at[pidx_v.at[j]],
                                       prows_v.at[sl], sem))
        copies.append(pltpu.async_copy(item_table.at[nidx_v.at[j]],
                                       nrows_v.at[sl], sem))
    for c in copies:
        c.wait()

    iota = lax.iota(jnp.int32, 16)
    bval = bv_v[:]
    wc = [wb_v[pl.ds(c * 16, 16)] for c in range(4)]

    def group(g, _):
        rbase = g * 16
        for r16 in range(16):
            rr = rbase + r16
            accp = jnp.zeros((16,), jnp.float32)
            accn = jnp.zeros((16,), jnp.float32)
            for c in range(4):
                csl = pl.ds(c * 16, 16)
                u = urows_v[rr, csl]
                p = prows_v[rr, csl]
                n = nrows_v[rr, csl]
                accp = accp + wc[c] / (1.0 + jnp.exp(-(u * p)))
                accn = accn + wc[c] / (1.0 + jnp.exp(-(u * n)))
            tsp_v[pl.ds(r16 * 16, 16)] = accp
            tsn_v[pl.ds(r16 * 16, 16)] = accn
        sump = bval
        sumn = bval
        cols = iota * 16
        for l in range(16):
            sump = sump + plsc.load_gather(tsp_v, [cols + l])
            sumn = sumn + plsc.load_gather(tsn_v, [cols + l])
        outp_v[pl.ds(rbase, 16)] = 1.0 / (1.0 + jnp.exp(-sump))
        outn_v[pl.ds(rbase, 16)] = 1.0 / (1.0 + jnp.exp(-sumn))
        return 0

    lax.fori_loop(0, GPW, group, 0)

    obase = wid * RPW
    pltpu.sync_copy(outp_v, outp_hbm.at[pl.ds(obase, RPW)])
    pltpu.sync_copy(outn_v, outn_hbm.at[pl.ds(obase, RPW)])


@jax.jit
def kernel(user, pos, neg, user_table, item_table, W, b):
    user2d = user.reshape(B // IDX_CHUNK, IDX_CHUNK)
    pos2d = pos.reshape(B // IDX_CHUNK, IDX_CHUNK)
    neg2d = neg.reshape(B // IDX_CHUNK, IDX_CHUNK)
    wb = W.reshape(D)
    bv = jnp.broadcast_to(b.reshape(1), (16,))

    mesh = plsc.VectorSubcoreMesh(core_axis_name="c", subcore_axis_name="s")
    run = pl.kernel(
        _mf_body,
        out_type=(jax.ShapeDtypeStruct((B,), jnp.float32),
                  jax.ShapeDtypeStruct((B,), jnp.float32)),
        mesh=mesh,
        compiler_params=pltpu.CompilerParams(needs_layout_passes=False,
                                              use_tc_tiling_on_sc=False),
        scratch_types=[
            pltpu.VMEM((NCHUNK, IDX_CHUNK), jnp.int32),
            pltpu.VMEM((NCHUNK, IDX_CHUNK), jnp.int32),
            pltpu.VMEM((NCHUNK, IDX_CHUNK), jnp.int32),
            pltpu.VMEM((RPW, D), jnp.float32),
            pltpu.VMEM((RPW, D), jnp.float32),
            pltpu.VMEM((RPW, D), jnp.float32),
            pltpu.VMEM((D,), jnp.float32),
            pltpu.VMEM((16,), jnp.float32),
            pltpu.VMEM((256,), jnp.float32),
            pltpu.VMEM((256,), jnp.float32),
            pltpu.VMEM((RPW,), jnp.float32),
            pltpu.VMEM((RPW,), jnp.float32),
            pltpu.SemaphoreType.DMA,
        ],
    )
    outp, outn = run(user2d, pos2d, neg2d, user_table, item_table, wb, bv)
    return jnp.stack([outp, outn], axis=1)
